# Initial kernel scaffold; baseline (speedup 1.0000x reference)
import jax
import jax.numpy as jnp
from jax.experimental import pallas as pl


def kernel(input, token_table, pos_table):
    raise NotImplementedError("write your pallas kernel here")


# speedup vs baseline: 1.2348x; 1.2348x over previous
import functools

import jax
import jax.numpy as jnp
from jax import lax
from jax.experimental import pallas as pl
from jax.experimental.pallas import tpu as pltpu
from jax.experimental.pallas import tpu_sc as plsc


def kernel(input, token_table, pos_table):
    B, S = input.shape
    V, D = token_table.shape
    N = B * S

    info = plsc.get_sparse_core_info()
    NC, NS = info.num_cores, info.num_subcores
    NW = NC * NS
    n_per_w = N // NW
    CH = S
    n_ch = n_per_w // CH

    idx = input.reshape(N)

    mesh = plsc.VectorSubcoreMesh(core_axis_name="c", subcore_axis_name="s")

    @functools.partial(
        pl.kernel,
        mesh=mesh,
        compiler_params=pltpu.CompilerParams(use_tc_tiling_on_sc=False),
        out_type=jax.ShapeDtypeStruct((N, D), jnp.float32),
        scratch_types=[
            pltpu.VMEM((n_per_w,), jnp.int32),
            pltpu.VMEM((S, D), jnp.float32),
            pltpu.VMEM((CH, D), jnp.float32),
            pltpu.SemaphoreType.DMA,
        ],
    )
    def emb(idx_hbm, tok_hbm, pos_hbm, out_hbm, idx_v, pos_v, buf, sem):
        wid = lax.axis_index("s") * NC + lax.axis_index("c")
        base = wid * n_per_w
        pltpu.sync_copy(idx_hbm.at[pl.ds(base, n_per_w)], idx_v)
        pltpu.sync_copy(pos_hbm, pos_v)

        def chunk_body(j, carry):
            pltpu.async_copy(
                tok_hbm.at[idx_v.at[pl.ds(j * CH, CH)]], buf, sem
            ).wait()

            def add_row(i, c):
                for h in range(D // 16):
                    sl = pl.ds(h * 16, 16)
                    buf[i, sl] = buf[i, sl] + pos_v[i, sl]
                return c

            lax.fori_loop(0, CH, add_row, 0)
            pltpu.sync_copy(buf, out_hbm.at[pl.ds(base + j * CH, CH)])
            return carry

        lax.fori_loop(0, n_ch, chunk_body, 0)

    out = emb(idx, token_table, pos_table)
    return out.reshape(B, S, D)


# speedup vs baseline: 1.4913x; 1.2078x over previous
import functools

import jax
import jax.numpy as jnp
from jax import lax
from jax.experimental import pallas as pl
from jax.experimental.pallas import tpu as pltpu
from jax.experimental.pallas import tpu_sc as plsc

NBUF = 4


def kernel(input, token_table, pos_table):
    B, S = input.shape
    V, D = token_table.shape
    N = B * S

    info = plsc.get_sparse_core_info()
    NC, NS = info.num_cores, info.num_subcores
    NW = NC * NS
    n_per_w = N // NW
    CH = S
    n_ch = n_per_w // CH
    T = n_ch // NBUF

    idx = input.reshape(N)

    mesh = plsc.VectorSubcoreMesh(core_axis_name="c", subcore_axis_name="s")

    @functools.partial(
        pl.kernel,
        mesh=mesh,
        compiler_params=pltpu.CompilerParams(use_tc_tiling_on_sc=False),
        out_type=jax.ShapeDtypeStruct((N, D), jnp.float32),
        scratch_types=(
            [pltpu.VMEM((n_per_w,), jnp.int32),
             pltpu.VMEM((S, D), jnp.float32)]
            + [pltpu.VMEM((CH, D), jnp.float32) for _ in range(NBUF)]
            + [pltpu.VMEM((CH, D), jnp.float32) for _ in range(NBUF)]
            + [pltpu.SemaphoreType.DMA for _ in range(2 * NBUF)]
        ),
    )
    def emb(idx_hbm, tok_hbm, pos_hbm, out_hbm, idx_v, pos_v, *rest):
        bufs = rest[:NBUF]
        obufs = rest[NBUF:2 * NBUF]
        gsems = rest[2 * NBUF:3 * NBUF]
        osems = rest[3 * NBUF:]

        wid = lax.axis_index("s") * NC + lax.axis_index("c")
        base = wid * n_per_w
        pltpu.sync_copy(idx_hbm.at[pl.ds(base, n_per_w)], idx_v)
        pltpu.sync_copy(pos_hbm, pos_v)

        def issue_gather(j, b):
            pltpu.async_copy(
                tok_hbm.at[idx_v.at[pl.ds(j * CH, CH)]], bufs[b], gsems[b]
            )

        def wait_gather(b):
            pltpu.make_async_copy(
                tok_hbm.at[pl.ds(0, CH)], bufs[b], gsems[b]
            ).wait()

        def compute(b):
            buf, obuf = bufs[b], obufs[b]

            @plsc.parallel_loop(0, CH, unroll=8)
            def add_row(i):
                for h in range(D // 16):
                    sl = pl.ds(h * 16, 16)
                    obuf[i, sl] = buf[i, sl] + pos_v[i, sl]

        def issue_out(j, b):
            pltpu.async_copy(
                obufs[b], out_hbm.at[pl.ds(base + j * CH, CH)], osems[b]
            )

        def wait_out(b):
            pltpu.make_async_copy(
                obufs[b], out_hbm.at[pl.ds(base, CH)], osems[b]
            ).wait()

        for b in range(NBUF):
            issue_gather(b, b)

        for b in range(NBUF):
            wait_gather(b)
            compute(b)
            issue_gather(NBUF + b, b)
            issue_out(b, b)

        @pl.loop(1, T - 1)
        def outer(t):
            for b in range(NBUF):
                j = t * NBUF + b
                wait_gather(b)
                wait_out(b)
                compute(b)
                issue_gather(j + NBUF, b)
                issue_out(j, b)

        for b in range(NBUF):
            j = (T - 1) * NBUF + b
            wait_gather(b)
            wait_out(b)
            compute(b)
            issue_out(j, b)

        for b in range(NBUF):
            wait_out(b)

    out = emb(idx, token_table, pos_table)
    return out.reshape(B, S, D)


# speedup vs baseline: 1.5495x; 1.0390x over previous
import functools

import jax
import jax.numpy as jnp
from jax import lax
from jax.experimental import pallas as pl
from jax.experimental.pallas import tpu as pltpu
from jax.experimental.pallas import tpu_sc as plsc

NBUF = 4


def kernel(input, token_table, pos_table):
    B, S = input.shape
    V, D = token_table.shape
    G = D // 8
    TB = B // 128

    inputT = input.T

    info = plsc.get_sparse_core_info()
    NC, NS = info.num_cores, info.num_subcores

    mesh = plsc.VectorSubcoreMesh(core_axis_name="c", subcore_axis_name="s")

    @functools.partial(
        pl.kernel,
        mesh=mesh,
        compiler_params=pltpu.CompilerParams(
            use_tc_tiling_on_sc=False, needs_layout_passes=False
        ),
        out_type=jax.ShapeDtypeStruct((S * G * TB, 1024), jnp.float32),
        scratch_types=(
            [pltpu.VMEM((S, 128), jnp.int32),
             pltpu.VMEM((S, D), jnp.float32)]
            + [pltpu.VMEM((128, D), jnp.float32) for _ in range(NBUF)]
            + [pltpu.VMEM((G * 1024,), jnp.float32) for _ in range(NBUF)]
            + [pltpu.SemaphoreType.DMA for _ in range(2 * NBUF)]
        ),
    )
    def emb(idxT_hbm, tok_hbm, pos_hbm, out_hbm, idx_v, pos_v, *rest):
        bufs = rest[:NBUF]
        obufs = rest[NBUF:2 * NBUF]
        gsems = rest[2 * NBUF:3 * NBUF]
        osems = rest[3 * NBUF:]

        w = lax.axis_index("s") * NC + lax.axis_index("c")

        @pl.loop(0, S)
        def stage(s):
            pltpu.sync_copy(
                idxT_hbm.at[s, pl.ds(w * 128, 128)], idx_v.at[s]
            )

        pltpu.sync_copy(pos_hbm, pos_v)

        def issue_gather(s, b):
            pltpu.async_copy(tok_hbm.at[idx_v.at[s]], bufs[b], gsems[b])

        def wait_gather(b):
            pltpu.make_async_copy(
                tok_hbm.at[pl.ds(0, 128)], bufs[b], gsems[b]
            ).wait()

        def compute(s, b):
            buf, obuf = bufs[b], obufs[b]
            lanes = jax.lax.broadcasted_iota(jnp.int32, (16,), 0)
            fidx = [((lanes + 16 * h) // 8) * 1024
                    + ((lanes + 16 * h) % 8) * 128 for h in range(2)]
            prow = [pos_v[s, pl.ds(16 * h, 16)] for h in range(2)]

            @plsc.parallel_loop(0, 128, unroll=8)
            def body(bb):
                for h in range(2):
                    x = buf[bb, pl.ds(16 * h, 16)] + prow[h]
                    plsc.store_scatter(obuf, [fidx[h] + bb], x)

        def issue_out(s, b):
            for g in range(G):
                row = (s * G + g) * TB + w
                pltpu.async_copy(
                    obufs[b].at[pl.ds(g * 1024, 1024)], out_hbm.at[row],
                    osems[b]
                )

        def wait_out(b):
            for g in range(G):
                pltpu.make_async_copy(
                    obufs[b].at[pl.ds(g * 1024, 1024)], out_hbm.at[g],
                    osems[b]
                ).wait()

        for b in range(NBUF):
            issue_gather(b, b)

        for b in range(NBUF):
            wait_gather(b)
            compute(b, b)
            issue_gather(NBUF + b, b)
            issue_out(b, b)

        T = S // NBUF

        @pl.loop(1, T - 1)
        def outer(t):
            for b in range(NBUF):
                s = t * NBUF + b
                wait_gather(b)
                wait_out(b)
                compute(s, b)
                issue_gather(s + NBUF, b)
                issue_out(s, b)

        for b in range(NBUF):
            s = (T - 1) * NBUF + b
            wait_gather(b)
            wait_out(b)
            compute(s, b)
            issue_out(s, b)

        for b in range(NBUF):
            wait_out(b)

    out5 = emb(inputT, token_table, pos_table)
    out = out5.reshape(S, G, TB, 8, 128).transpose(2, 4, 0, 1, 3)
    return out.reshape(B, S, D)


# speedup vs baseline: 1.7186x; 1.1091x over previous
import functools

import jax
import jax.numpy as jnp
from jax import lax
from jax.experimental import pallas as pl
from jax.experimental.pallas import tpu as pltpu
from jax.experimental.pallas import tpu_sc as plsc

NBUF = 4


def kernel(input, token_table, pos_table):
    B, S = input.shape
    V, D = token_table.shape
    G = D // 8
    TB = B // 128

    inputT = input.T

    info = plsc.get_sparse_core_info()
    NC, NS = info.num_cores, info.num_subcores

    mesh = plsc.VectorSubcoreMesh(core_axis_name="c", subcore_axis_name="s")

    @functools.partial(
        pl.kernel,
        mesh=mesh,
        compiler_params=pltpu.CompilerParams(
            use_tc_tiling_on_sc=False, needs_layout_passes=False
        ),
        out_type=jax.ShapeDtypeStruct((S, G, TB, 1024), jnp.float32),
        scratch_types=(
            [pltpu.VMEM((S, 128), jnp.int32),
             pltpu.VMEM((S, D), jnp.float32)]
            + [pltpu.VMEM((128, D), jnp.float32) for _ in range(NBUF)]
            + [pltpu.VMEM((G, 1024), jnp.float32) for _ in range(NBUF)]
            + [pltpu.SemaphoreType.DMA for _ in range(2 * NBUF)]
        ),
    )
    def emb(idxT_hbm, tok_hbm, pos_hbm, out_hbm, idx_v, pos_v, *rest):
        bufs = rest[:NBUF]
        obufs = rest[NBUF:2 * NBUF]
        gsems = rest[2 * NBUF:3 * NBUF]
        osems = rest[3 * NBUF:]

        w = lax.axis_index("s") * NC + lax.axis_index("c")

        pltpu.sync_copy(idxT_hbm.at[:, pl.ds(w * 128, 128)], idx_v)

        pltpu.sync_copy(pos_hbm, pos_v)

        def issue_gather(s, b):
            pltpu.async_copy(tok_hbm.at[idx_v.at[s]], bufs[b], gsems[b])

        def wait_gather(b):
            pltpu.make_async_copy(
                tok_hbm.at[pl.ds(0, 128)], bufs[b], gsems[b]
            ).wait()

        def compute(s, b):
            buf, obuf = bufs[b], obufs[b]
            lanes = jax.lax.broadcasted_iota(jnp.int32, (16,), 0)
            gidx = [(lanes + 16 * h) // 8 for h in range(2)]
            ridx = [((lanes + 16 * h) % 8) * 128 for h in range(2)]
            prow = [pos_v[s, pl.ds(16 * h, 16)] for h in range(2)]

            @plsc.parallel_loop(0, 128, unroll=8)
            def body(bb):
                for h in range(2):
                    x = buf[bb, pl.ds(16 * h, 16)] + prow[h]
                    plsc.store_scatter(obuf, [gidx[h], ridx[h] + bb], x)

        def issue_out(s, b):
            pltpu.async_copy(
                obufs[b], out_hbm.at[s, :, w], osems[b]
            )

        def wait_out(b):
            pltpu.make_async_copy(
                obufs[b], out_hbm.at[0, :, 0], osems[b]
            ).wait()

        for b in range(NBUF):
            issue_gather(b, b)

        for b in range(NBUF):
            wait_gather(b)
            compute(b, b)
            issue_gather(NBUF + b, b)
            issue_out(b, b)

        T = S // NBUF

        @pl.loop(1, T - 1)
        def outer(t):
            for b in range(NBUF):
                s = t * NBUF + b
                wait_gather(b)
                wait_out(b)
                compute(s, b)
                issue_gather(s + NBUF, b)
                issue_out(s, b)

        for b in range(NBUF):
            s = (T - 1) * NBUF + b
            wait_gather(b)
            wait_out(b)
            compute(s, b)
            issue_out(s, b)

        for b in range(NBUF):
            wait_out(b)

    out5 = emb(inputT, token_table, pos_table)
    out = out5.reshape(S, G, TB, 8, 128).transpose(2, 4, 0, 1, 3)
    return out.reshape(B, S, D)


# speedup vs baseline: 2.4646x; 1.4341x over previous
import functools

import jax
import jax.numpy as jnp
from jax import lax
from jax.experimental import pallas as pl
from jax.experimental.pallas import tpu as pltpu
from jax.experimental.pallas import tpu_sc as plsc

NBUF = 4
PITCH = 33


def kernel(input, token_table, pos_table):
    B, S = input.shape
    V, D = token_table.shape
    G = D // 8
    TB = B // 128
    SG = S // 8

    inp4 = input.reshape(TB, 128, SG, 8).transpose(2, 0, 3, 1)

    info = plsc.get_sparse_core_info()
    NC, NS = info.num_cores, info.num_subcores

    mesh = plsc.VectorSubcoreMesh(core_axis_name="c", subcore_axis_name="s")

    @functools.partial(
        pl.kernel,
        mesh=mesh,
        compiler_params=pltpu.CompilerParams(
            use_tc_tiling_on_sc=False, needs_layout_passes=False
        ),
        out_type=jax.ShapeDtypeStruct((S, G, TB, 1024), jnp.float32),
        scratch_types=(
            [pltpu.VMEM((SG, 8, 128), jnp.int32),
             pltpu.VMEM((S, D), jnp.float32)]
            + [pltpu.VMEM((128, D), jnp.float32) for _ in range(NBUF)]
            + [pltpu.VMEM((128, PITCH), jnp.float32) for _ in range(NBUF)]
            + [pltpu.VMEM((G, 1024), jnp.float32) for _ in range(NBUF)]
            + [pltpu.SemaphoreType.DMA for _ in range(2 * NBUF)]
        ),
    )
    def emb(idx_hbm, tok_hbm, pos_hbm, out_hbm, idx_v, pos_v, *rest):
        bufs = rest[:NBUF]
        pbufs = rest[NBUF:2 * NBUF]
        obufs = rest[2 * NBUF:3 * NBUF]
        gsems = rest[3 * NBUF:4 * NBUF]
        osems = rest[4 * NBUF:]

        w = lax.axis_index("s") * NC + lax.axis_index("c")

        pltpu.sync_copy(idx_hbm.at[:, w], idx_v)
        pltpu.sync_copy(pos_hbm, pos_v)

        lanes = jax.lax.broadcasted_iota(jnp.int32, (16,), 0)

        def issue_gather(s, b):
            pltpu.async_copy(
                tok_hbm.at[idx_v.at[s // 8, s % 8]], bufs[b], gsems[b]
            )

        def wait_gather(b):
            pltpu.make_async_copy(
                tok_hbm.at[pl.ds(0, 128)], bufs[b], gsems[b]
            ).wait()

        def compute(s, b):
            buf, pbuf, obuf = bufs[b], pbufs[b], obufs[b]
            rvecs = [bb0 * 16 + lanes for bb0 in range(8)]
            prow = [pos_v[s, pl.ds(16 * h, 16)] for h in range(2)]

            @plsc.parallel_loop(0, 128, unroll=8)
            def rloop(r):
                for h in range(2):
                    pbuf[r, pl.ds(16 * h, 16)] = (
                        buf[r, pl.ds(16 * h, 16)] + prow[h]
                    )

            for h in range(2):
                @plsc.parallel_loop(0, 16, unroll=4)
                def dloop(dq):
                    d = 16 * h + dq
                    cols = jnp.zeros((16,), jnp.int32) + d
                    g = d // 8
                    off = (d % 8) * 128
                    for bb0 in range(8):
                        val = plsc.load_gather(pbuf, [rvecs[bb0], cols])
                        obuf[g, pl.ds(off + bb0 * 16, 16)] = val

        def issue_out(s, b):
            pltpu.async_copy(obufs[b], out_hbm.at[s, :, w], osems[b])

        def wait_out(b):
            pltpu.make_async_copy(
                obufs[b], out_hbm.at[0, :, 0], osems[b]
            ).wait()

        for b in range(NBUF):
            issue_gather(b, b)

        for b in range(NBUF):
            wait_gather(b)
            compute(b, b)
            issue_gather(NBUF + b, b)
            issue_out(b, b)

        T = S // NBUF

        @pl.loop(1, T - 1)
        def outer(t):
            for b in range(NBUF):
                s = t * NBUF + b
                wait_gather(b)
                wait_out(b)
                compute(s, b)
                issue_gather(s + NBUF, b)
                issue_out(s, b)

        for b in range(NBUF):
            s = (T - 1) * NBUF + b
            wait_gather(b)
            wait_out(b)
            compute(s, b)
            issue_out(s, b)

        for b in range(NBUF):
            wait_out(b)

    out5 = emb(inp4, token_table, pos_table)
    out = out5.reshape(S, G, TB, 8, 128).transpose(2, 4, 0, 1, 3)
    return out.reshape(B, S, D)
